# baseline (speedup 1.0000x reference)
import functools

import jax
import jax.numpy as jnp
from jax import lax
from jax.experimental import pallas as pl
from jax.experimental.pallas import tpu as pltpu
from jax.experimental.pallas import tpu_sc as plsc

N = 10000
E = 320000
NUM_GRAPHS = 256
NUM_ELEMENTS = 100
NUM_OXIDATIONS = 16
NUM_GEOMETRIES = 64
F = 256
FH = 128
NP = 10240
R = 1000
G = N // R
CIN = 184


def _elu(x):
    return jnp.where(x > 0, x, jnp.exp(jnp.minimum(x, 0.0)) - 1.0)


def _mat1_body(elem_ref, ox_ref, geo_ref, ang_ref, deg_ref, w_ref,
               dinv_ref, h0_ref, h1_ref):
    cls = lax.broadcasted_iota(jnp.int32, (1, CIN), 1)
    oh = ((elem_ref[0] == cls).astype(jnp.float32)
          + (ox_ref[0] + NUM_ELEMENTS == cls).astype(jnp.float32)
          + (geo_ref[0] + NUM_ELEMENTS + NUM_OXIDATIONS == cls).astype(jnp.float32))
    a0 = ang_ref[:, 0:1] * (cls == 180).astype(jnp.float32)
    a1 = ang_ref[:, 1:2] * (cls == 181).astype(jnp.float32)
    x_in = oh + a0 + a1
    deg = deg_ref[0][:, 0:1] + deg_ref[1][:, 0:1]
    dinv = lax.rsqrt(deg)
    h = jnp.dot(x_in, w_ref[...], preferred_element_type=jnp.float32) * dinv
    dinv_ref[...] = jnp.broadcast_to(dinv, (R, 8))
    h0_ref[...] = h[:, :FH]
    h1_ref[...] = h[:, FH:]


def _mat1(elem3, ox3, geo3, angles, deg2, w1eff):
    return pl.pallas_call(
        _mat1_body,
        grid=(G,),
        in_specs=[
            pl.BlockSpec((1, R, 1), lambda i: (i, 0, 0)),
            pl.BlockSpec((1, R, 1), lambda i: (i, 0, 0)),
            pl.BlockSpec((1, R, 1), lambda i: (i, 0, 0)),
            pl.BlockSpec((R, 2), lambda i: (i, 0)),
            pl.BlockSpec((2, R, 16), lambda i: (0, i, 0)),
            pl.BlockSpec((CIN, F), lambda i: (0, 0)),
        ],
        out_specs=[
            pl.BlockSpec((R, 8), lambda i: (i, 0)),
            pl.BlockSpec((R, FH), lambda i: (i, 0)),
            pl.BlockSpec((R, FH), lambda i: (i, 0)),
        ],
        out_shape=[
            jax.ShapeDtypeStruct((N, 8), jnp.float32),
            jax.ShapeDtypeStruct((N, FH), jnp.float32),
            jax.ShapeDtypeStruct((N, FH), jnp.float32),
        ],
    )(elem3, ox3, geo3, angles, deg2, w1eff)


def _mat2_body(acc_ref, dinv_ref, b_ref, w_ref, h0_ref, h1_ref):
    xcat = jnp.concatenate([acc_ref[0], acc_ref[1]], axis=1)
    dinv = dinv_ref[:, 0:1]
    x = _elu(xcat * dinv + b_ref[...])
    h = jnp.dot(x, w_ref[...], preferred_element_type=jnp.float32) * dinv
    h0_ref[...] = h[:, :FH]
    h1_ref[...] = h[:, FH:]


def _mat2(acc2, dinv8, bprev, w):
    return pl.pallas_call(
        _mat2_body,
        grid=(G,),
        in_specs=[
            pl.BlockSpec((2, R, FH), lambda i: (0, i, 0)),
            pl.BlockSpec((R, 8), lambda i: (i, 0)),
            pl.BlockSpec((1, F), lambda i: (0, 0)),
            pl.BlockSpec((F, F), lambda i: (0, 0)),
        ],
        out_specs=[
            pl.BlockSpec((R, FH), lambda i: (i, 0)),
            pl.BlockSpec((R, FH), lambda i: (i, 0)),
        ],
        out_shape=[
            jax.ShapeDtypeStruct((N, FH), jnp.float32),
            jax.ShapeDtypeStruct((N, FH), jnp.float32),
        ],
    )(acc2, dinv8, bprev, w)


def _head_body(acc_ref, dinv_ref, b3_ref, batch_ref, dw1_ref, db1_ref,
               dw2_ref, db2_ref, dw3_ref, db3_ref, out_ref, sums_ref, cnt_ref):
    i = pl.program_id(0)
    xcat = jnp.concatenate([acc_ref[0], acc_ref[1]], axis=1)
    x3 = _elu(xcat * dinv_ref[:, 0:1] + b3_ref[...])
    gcls = lax.broadcasted_iota(jnp.int32, (1, NUM_GRAPHS), 1)
    oh = (batch_ref[0] == gcls).astype(jnp.float32)
    dn = (((0,), (0,)), ((), ()))
    s = lax.dot_general(oh, x3, dn, preferred_element_type=jnp.float32)
    c = lax.dot_general(oh, jnp.ones((R, 8), jnp.float32), dn,
                        preferred_element_type=jnp.float32)

    @pl.when(i == 0)
    def _():
        sums_ref[...] = s
        cnt_ref[...] = c

    @pl.when(i > 0)
    def _():
        sums_ref[...] += s
        cnt_ref[...] += c

    @pl.when(i == G - 1)
    def _():
        pooled = sums_ref[...] / jnp.maximum(cnt_ref[:, 0:1], 1.0)
        h1 = _elu(jnp.dot(pooled, dw1_ref[...],
                          preferred_element_type=jnp.float32) + db1_ref[...])
        h2 = _elu(jnp.dot(h1, dw2_ref[...],
                          preferred_element_type=jnp.float32) + db2_ref[...])
        out_ref[...] = jnp.dot(h2, dw3_ref[...],
                               preferred_element_type=jnp.float32) + db3_ref[...]


def _head(acc2, dinv8, b3, batch3, dw1, db1, dw2, db2, dw3p, db3p):
    return pl.pallas_call(
        _head_body,
        grid=(G,),
        in_specs=[
            pl.BlockSpec((2, R, FH), lambda i: (0, i, 0)),
            pl.BlockSpec((R, 8), lambda i: (i, 0)),
            pl.BlockSpec((1, F), lambda i: (0, 0)),
            pl.BlockSpec((1, R, 1), lambda i: (i, 0, 0)),
            pl.BlockSpec((F, 512), lambda i: (0, 0)),
            pl.BlockSpec((1, 512), lambda i: (0, 0)),
            pl.BlockSpec((512, 128), lambda i: (0, 0)),
            pl.BlockSpec((1, 128), lambda i: (0, 0)),
            pl.BlockSpec((128, 8), lambda i: (0, 0)),
            pl.BlockSpec((1, 8), lambda i: (0, 0)),
        ],
        out_specs=pl.BlockSpec((NUM_GRAPHS, 8), lambda i: (0, 0)),
        out_shape=jax.ShapeDtypeStruct((NUM_GRAPHS, 8), jnp.float32),
        scratch_shapes=[
            pltpu.VMEM((NUM_GRAPHS, F), jnp.float32),
            pltpu.VMEM((NUM_GRAPHS, 8), jnp.float32),
        ],
    )(acc2, dinv8, b3, batch3, dw1, db1, dw2, db2, dw3p, db3p)


EP = 344064
CH = 128
TE = EP // 16
KE = TE // CH
GZ = 24
NG = KE // GZ
TD = EP // 32
KD = TD // CH
RT = NP // 16
_SC_MESH = dict(core_axis_name="c", subcore_axis_name="s")


@functools.partial(
    pl.kernel,
    mesh=plsc.VectorSubcoreMesh(**_SC_MESH),
    out_type=jax.ShapeDtypeStruct((2, NP, 16), jnp.float32),
    scratch_types=[
        pltpu.VMEM_SHARED((NP, 16), jnp.float32),
        pltpu.VMEM((RT, 16), jnp.float32),
        pltpu.VMEM((CH, 16), jnp.float32),
        pltpu.VMEM((CH,), jnp.int32),
    ],
)
def _deg_kernel(dst_hbm, out_hbm, acc_sh, zbuf, ones, didx):
    c = lax.axis_index("c")
    s = lax.axis_index("s")

    def zrow(j, _):
        zbuf[j, :] = jnp.zeros((16,), jnp.float32)
        return 0
    lax.fori_loop(0, RT, zrow, 0)
    pltpu.sync_copy(zbuf, acc_sh.at[pl.ds(s * RT, RT)])

    def orow(j, _):
        ones[j, :] = jnp.full((16,), 1.0, jnp.float32)
        return 0
    lax.fori_loop(0, CH, orow, 0)
    plsc.subcore_barrier()

    w = c * 16 + s

    def body(k, _):
        base = w * TD + k * CH
        pltpu.sync_copy(dst_hbm.at[pl.ds(base, CH)], didx)
        pltpu.sync_copy(ones, acc_sh.at[didx], add=True)
        return 0
    lax.fori_loop(0, KD, body, 0)
    plsc.subcore_barrier()
    pltpu.sync_copy(acc_sh.at[pl.ds(s * RT, RT)],
                    out_hbm.at[c, pl.ds(s * RT, RT)])


def _deg_pass(src_pad, dst_pad):
    return _deg_kernel(dst_pad)


@functools.partial(
    pl.kernel,
    mesh=plsc.VectorSubcoreMesh(**_SC_MESH),
    out_type=jax.ShapeDtypeStruct((2, NP, FH), jnp.float32),
    scratch_types=[
        pltpu.VMEM_SHARED((NP, FH), jnp.float32),
        pltpu.VMEM((16, FH), jnp.float32),
        pltpu.VMEM((CH, FH), jnp.float32),
        pltpu.VMEM((CH, FH), jnp.float32),
        pltpu.VMEM((GZ, CH), jnp.int32),
        pltpu.VMEM((GZ, CH), jnp.int32),
        pltpu.SemaphoreType.DMA,
        pltpu.SemaphoreType.DMA,
        pltpu.SemaphoreType.DMA,
    ],
)
def _edge_kernel(h0_hbm, h1_hbm, src2_hbm, dst2_hbm, out_hbm,
                 acc_sh, zbuf, rows0, rows1, sidxb, didxb,
                 gsem, ssem0, ssem1):
    c = lax.axis_index("c")
    s = lax.axis_index("s")
    rows = (rows0, rows1)
    ssem = (ssem0, ssem1)

    def zrow(j, _):
        for q in range(FH // 16):
            zbuf[j, pl.ds(q * 16, 16)] = jnp.zeros((16,), jnp.float32)
        return 0
    lax.fori_loop(0, 16, zrow, 0)

    def zcp(p, _):
        pltpu.sync_copy(zbuf, acc_sh.at[pl.ds(s * RT + p * 16, 16)])
        return 0
    lax.fori_loop(0, RT // 16, zcp, 0)
    plsc.subcore_barrier()

    rb = s * KE

    def gath(idxref, rowsref):
        @pl.when(c == 0)
        def _():
            pltpu.async_copy(h0_hbm.at[idxref], rowsref, gsem)

        @pl.when(c == 1)
        def _():
            pltpu.async_copy(h1_hbm.at[idxref], rowsref, gsem)

    def gwait(rowsref):
        pltpu.make_async_copy(h0_hbm.at[sidxb.at[0]], rowsref, gsem).wait()

    def swait(p):
        pltpu.make_async_copy(rows[p], acc_sh.at[didxb.at[p]], ssem[p]).wait()

    def group(g, _):
        @pl.when(g > 0)
        def _():
            swait(0)
            swait(1)
        pltpu.sync_copy(src2_hbm.at[pl.ds(rb + g * GZ, GZ)], sidxb)
        pltpu.sync_copy(dst2_hbm.at[pl.ds(rb + g * GZ, GZ)], didxb)
        gath(sidxb.at[0], rows0)
        for j in range(GZ):
            p = j % 2
            gwait(rows[p])
            pltpu.async_copy(rows[p], acc_sh.at[didxb.at[j]], ssem[p],
                             add=True)
            if j + 1 < GZ:
                if j >= 1:
                    swait(1 - p)
                gath(sidxb.at[j + 1], rows[1 - p])
        return 0
    lax.fori_loop(0, NG, group, 0)
    swait(0)
    swait(1)
    plsc.subcore_barrier()
    pltpu.sync_copy(acc_sh.at[pl.ds(s * RT, RT)],
                    out_hbm.at[c, pl.ds(s * RT, RT)])


def _edge_pass(h0, h1, src2d, dst2d):
    return _edge_kernel(h0, h1, src2d, dst2d)


def kernel(elements, oxidations, geometries, angles, edge_index, batch,
           emb_elem, emb_ox, emb_geo,
           gcn_w1, gcn_b1, gcn_w2, gcn_b2, gcn_w3, gcn_b3,
           dense_w1, dense_b1, dense_w2, dense_b2, dense_w3, dense_b3):
    elem3 = elements.astype(jnp.int32).reshape(G, R, 1)
    ox3 = oxidations.astype(jnp.int32).reshape(G, R, 1)
    geo3 = geometries.astype(jnp.int32).reshape(G, R, 1)
    batch3 = batch.astype(jnp.int32).reshape(G, R, 1)
    angles = angles.astype(jnp.float32)

    w1p = jnp.pad(gcn_w1, ((0, 34), (0, 34)))
    b2d = jnp.zeros((CIN, F), jnp.float32)
    b2d = b2d.at[:NUM_ELEMENTS, :200].set(emb_elem)
    b2d = b2d.at[NUM_ELEMENTS:116, 200:210].set(emb_ox)
    b2d = b2d.at[116:180, 210:220].set(emb_geo)
    b2d = b2d.at[180, 220].set(1.0).at[181, 221].set(1.0)
    w1eff = b2d @ w1p

    w2p = jnp.pad(gcn_w2, ((0, 34), (0, 34)))
    w3p = jnp.pad(gcn_w3, ((0, 34), (0, 34)))
    b1p = jnp.pad(gcn_b1, (0, 34)).reshape(1, F)
    b2p = jnp.pad(gcn_b2, (0, 34)).reshape(1, F)
    b3p = jnp.pad(gcn_b3, (0, 34)).reshape(1, F)
    dw1p = jnp.pad(dense_w1, ((0, 34), (0, 0)))
    db1 = dense_b1.reshape(1, 512)
    db2 = dense_b2.reshape(1, 128)
    dw3p = jnp.pad(dense_w3, ((0, 0), (0, 7)))
    db3p = jnp.pad(dense_b3, (0, 7)).reshape(1, 8)

    src_pad = jnp.full((EP,), 0, jnp.int32)
    src_pad = src_pad.at[:E].set(edge_index[0].astype(jnp.int32))
    src_pad = src_pad.at[E:E + N].set(jnp.arange(N, dtype=jnp.int32))
    dst_pad = N + (jnp.arange(EP, dtype=jnp.int32) % (NP - N))
    dst_pad = dst_pad.at[:E].set(edge_index[1].astype(jnp.int32))
    dst_pad = dst_pad.at[E:E + N].set(jnp.arange(N, dtype=jnp.int32))
    src2d = src_pad.reshape(EP // CH, CH)
    dst2d = dst_pad.reshape(EP // CH, CH)

    deg2 = _deg_pass(src_pad, dst_pad)
    dinv8, h0, h1 = _mat1(elem3, ox3, geo3, angles, deg2, w1eff)
    acc2 = _edge_pass(h0, h1, src2d, dst2d)
    h0, h1 = _mat2(acc2, dinv8, b1p, w2p)
    acc2 = _edge_pass(h0, h1, src2d, dst2d)
    h0, h1 = _mat2(acc2, dinv8, b2p, w3p)
    acc2 = _edge_pass(h0, h1, src2d, dst2d)
    out8 = _head(acc2, dinv8, b3p, batch3, dw1p, db1, dense_w2, db2, dw3p, db3p)
    return out8[:, :1]


# speedup vs baseline: 2.6771x; 2.6771x over previous
import functools

import jax
import jax.numpy as jnp
from jax import lax
from jax.experimental import pallas as pl
from jax.experimental.pallas import tpu as pltpu
from jax.experimental.pallas import tpu_sc as plsc

N = 10000
E = 320000
NUM_GRAPHS = 256
NUM_ELEMENTS = 100
NUM_OXIDATIONS = 16
NUM_GEOMETRIES = 64
F = 256
FH = 128
NP = 10112
R = 1000
G = N // R
CIN = 184


def _elu(x):
    return jnp.where(x > 0, x, jnp.exp(jnp.minimum(x, 0.0)) - 1.0)


def _mat1_body(elem_ref, ox_ref, geo_ref, ang_ref, deg_ref, w_ref,
               dinv_ref, h0_ref, h1_ref):
    cls = lax.broadcasted_iota(jnp.int32, (1, CIN), 1)
    oh = ((elem_ref[0] == cls).astype(jnp.float32)
          + (ox_ref[0] + NUM_ELEMENTS == cls).astype(jnp.float32)
          + (geo_ref[0] + NUM_ELEMENTS + NUM_OXIDATIONS == cls).astype(jnp.float32))
    a0 = ang_ref[:, 0:1] * (cls == 180).astype(jnp.float32)
    a1 = ang_ref[:, 1:2] * (cls == 181).astype(jnp.float32)
    x_in = oh + a0 + a1
    deg = deg_ref[0][:, 0:1] + deg_ref[1][:, 0:1]
    dinv = lax.rsqrt(deg)
    h = jnp.dot(x_in, w_ref[...], preferred_element_type=jnp.float32) * dinv
    dinv_ref[...] = jnp.broadcast_to(dinv, (R, 8))
    h0_ref[...] = h[:, :FH]
    h1_ref[...] = h[:, FH:]


def _mat1(elem3, ox3, geo3, angles, deg2, w1eff):
    return pl.pallas_call(
        _mat1_body,
        grid=(G,),
        in_specs=[
            pl.BlockSpec((1, R, 1), lambda i: (i, 0, 0)),
            pl.BlockSpec((1, R, 1), lambda i: (i, 0, 0)),
            pl.BlockSpec((1, R, 1), lambda i: (i, 0, 0)),
            pl.BlockSpec((R, 2), lambda i: (i, 0)),
            pl.BlockSpec((2, R, 16), lambda i: (0, i, 0)),
            pl.BlockSpec((CIN, F), lambda i: (0, 0)),
        ],
        out_specs=[
            pl.BlockSpec((R, 8), lambda i: (i, 0)),
            pl.BlockSpec((R, FH), lambda i: (i, 0)),
            pl.BlockSpec((R, FH), lambda i: (i, 0)),
        ],
        out_shape=[
            jax.ShapeDtypeStruct((N, 8), jnp.float32),
            jax.ShapeDtypeStruct((N, FH), jnp.float32),
            jax.ShapeDtypeStruct((N, FH), jnp.float32),
        ],
    )(elem3, ox3, geo3, angles, deg2, w1eff)


def _mat2_body(acc_ref, dinv_ref, b_ref, w_ref, h0_ref, h1_ref):
    xcat = jnp.concatenate([acc_ref[0], acc_ref[1]], axis=1)
    dinv = dinv_ref[:, 0:1]
    x = _elu(xcat * dinv + b_ref[...])
    h = jnp.dot(x, w_ref[...], preferred_element_type=jnp.float32) * dinv
    h0_ref[...] = h[:, :FH]
    h1_ref[...] = h[:, FH:]


def _mat2(acc2, dinv8, bprev, w):
    return pl.pallas_call(
        _mat2_body,
        grid=(G,),
        in_specs=[
            pl.BlockSpec((2, R, FH), lambda i: (0, i, 0)),
            pl.BlockSpec((R, 8), lambda i: (i, 0)),
            pl.BlockSpec((1, F), lambda i: (0, 0)),
            pl.BlockSpec((F, F), lambda i: (0, 0)),
        ],
        out_specs=[
            pl.BlockSpec((R, FH), lambda i: (i, 0)),
            pl.BlockSpec((R, FH), lambda i: (i, 0)),
        ],
        out_shape=[
            jax.ShapeDtypeStruct((N, FH), jnp.float32),
            jax.ShapeDtypeStruct((N, FH), jnp.float32),
        ],
    )(acc2, dinv8, bprev, w)


def _head_body(acc_ref, dinv_ref, b3_ref, batch_ref, dw1_ref, db1_ref,
               dw2_ref, db2_ref, dw3_ref, db3_ref, out_ref, sums_ref, cnt_ref):
    i = pl.program_id(0)
    xcat = jnp.concatenate([acc_ref[0], acc_ref[1]], axis=1)
    x3 = _elu(xcat * dinv_ref[:, 0:1] + b3_ref[...])
    gcls = lax.broadcasted_iota(jnp.int32, (1, NUM_GRAPHS), 1)
    oh = (batch_ref[0] == gcls).astype(jnp.float32)
    dn = (((0,), (0,)), ((), ()))
    s = lax.dot_general(oh, x3, dn, preferred_element_type=jnp.float32)
    c = lax.dot_general(oh, jnp.ones((R, 8), jnp.float32), dn,
                        preferred_element_type=jnp.float32)

    @pl.when(i == 0)
    def _():
        sums_ref[...] = s
        cnt_ref[...] = c

    @pl.when(i > 0)
    def _():
        sums_ref[...] += s
        cnt_ref[...] += c

    @pl.when(i == G - 1)
    def _():
        pooled = sums_ref[...] / jnp.maximum(cnt_ref[:, 0:1], 1.0)
        h1 = _elu(jnp.dot(pooled, dw1_ref[...],
                          preferred_element_type=jnp.float32) + db1_ref[...])
        h2 = _elu(jnp.dot(h1, dw2_ref[...],
                          preferred_element_type=jnp.float32) + db2_ref[...])
        out_ref[...] = jnp.dot(h2, dw3_ref[...],
                               preferred_element_type=jnp.float32) + db3_ref[...]


def _head(acc2, dinv8, b3, batch3, dw1, db1, dw2, db2, dw3p, db3p):
    return pl.pallas_call(
        _head_body,
        grid=(G,),
        in_specs=[
            pl.BlockSpec((2, R, FH), lambda i: (0, i, 0)),
            pl.BlockSpec((R, 8), lambda i: (i, 0)),
            pl.BlockSpec((1, F), lambda i: (0, 0)),
            pl.BlockSpec((1, R, 1), lambda i: (i, 0, 0)),
            pl.BlockSpec((F, 512), lambda i: (0, 0)),
            pl.BlockSpec((1, 512), lambda i: (0, 0)),
            pl.BlockSpec((512, 128), lambda i: (0, 0)),
            pl.BlockSpec((1, 128), lambda i: (0, 0)),
            pl.BlockSpec((128, 8), lambda i: (0, 0)),
            pl.BlockSpec((1, 8), lambda i: (0, 0)),
        ],
        out_specs=pl.BlockSpec((NUM_GRAPHS, 8), lambda i: (0, 0)),
        out_shape=jax.ShapeDtypeStruct((NUM_GRAPHS, 8), jnp.float32),
        scratch_shapes=[
            pltpu.VMEM((NUM_GRAPHS, F), jnp.float32),
            pltpu.VMEM((NUM_GRAPHS, 8), jnp.float32),
        ],
    )(acc2, dinv8, b3, batch3, dw1, db1, dw2, db2, dw3p, db3p)


EP = 331776
CH = 128
TE = EP // 16
KE = TE // CH
TD = EP // 32
KD = TD // CH
RT = NP // 16
_SC_MESH = dict(core_axis_name="c", subcore_axis_name="s")


@functools.partial(
    pl.kernel,
    mesh=plsc.VectorSubcoreMesh(**_SC_MESH),
    out_type=jax.ShapeDtypeStruct((2, NP, 16), jnp.float32),
    scratch_types=[
        pltpu.VMEM_SHARED((NP, 16), jnp.float32),
        pltpu.VMEM((RT, 16), jnp.float32),
        pltpu.VMEM((CH, 16), jnp.float32),
        pltpu.VMEM((CH,), jnp.int32),
    ],
)
def _deg_kernel(dst_hbm, out_hbm, acc_sh, zbuf, ones, didx):
    c = lax.axis_index("c")
    s = lax.axis_index("s")

    def zrow(j, _):
        zbuf[j, :] = jnp.zeros((16,), jnp.float32)
        return 0
    lax.fori_loop(0, RT, zrow, 0)
    pltpu.sync_copy(zbuf, acc_sh.at[pl.ds(s * RT, RT)])

    def orow(j, _):
        ones[j, :] = jnp.full((16,), 1.0, jnp.float32)
        return 0
    lax.fori_loop(0, CH, orow, 0)
    plsc.subcore_barrier()

    w = c * 16 + s

    def body(k, _):
        base = w * TD + k * CH
        pltpu.sync_copy(dst_hbm.at[pl.ds(base, CH)], didx)
        pltpu.sync_copy(ones, acc_sh.at[didx], add=True)
        return 0
    lax.fori_loop(0, KD, body, 0)
    plsc.subcore_barrier()
    pltpu.sync_copy(acc_sh.at[pl.ds(s * RT, RT)],
                    out_hbm.at[c, pl.ds(s * RT, RT)])


def _deg_pass(src_pad, dst_pad):
    return _deg_kernel(dst_pad)


@functools.partial(
    pl.kernel,
    mesh=plsc.VectorSubcoreMesh(**_SC_MESH),
    out_type=jax.ShapeDtypeStruct((2, NP, FH), jnp.float32),
    scratch_types=[
        pltpu.VMEM_SHARED((NP, FH), jnp.float32),
        pltpu.VMEM((CH, FH), jnp.float32),
        pltpu.VMEM((CH, FH), jnp.float32),
        pltpu.VMEM((CH, FH), jnp.float32),
        pltpu.VMEM((CH,), jnp.int32),
        pltpu.VMEM((CH,), jnp.int32),
        pltpu.VMEM((CH,), jnp.int32),
        pltpu.VMEM((CH,), jnp.int32),
        pltpu.VMEM((CH,), jnp.int32),
        pltpu.VMEM((CH,), jnp.int32),
        pltpu.SemaphoreType.DMA,
        pltpu.SemaphoreType.DMA,
        pltpu.SemaphoreType.DMA,
        pltpu.SemaphoreType.DMA,
        pltpu.SemaphoreType.DMA,
        pltpu.SemaphoreType.DMA,
    ],
)
def _edge_kernel(h0_hbm, h1_hbm, src_hbm, dst_hbm, zero_hbm, out_hbm,
                 acc_sh, rows0, rows1, rows2,
                 sidx0, sidx1, sidx2, didx0, didx1, didx2,
                 gsem0, gsem1, gsem2, ssem0, ssem1, ssem2):
    c = lax.axis_index("c")
    s = lax.axis_index("s")
    rows = (rows0, rows1, rows2)
    sidx = (sidx0, sidx1, sidx2)
    didx = (didx0, didx1, didx2)
    gsem = (gsem0, gsem1, gsem2)
    ssem = (ssem0, ssem1, ssem2)

    pltpu.sync_copy(zero_hbm.at[pl.ds(s * RT, RT)],
                    acc_sh.at[pl.ds(s * RT, RT)])
    plsc.subcore_barrier()

    tb = s * TE

    def gath(p, k):
        pltpu.sync_copy(src_hbm.at[pl.ds(tb + k * CH, CH)], sidx[p])

        @pl.when(c == 0)
        def _():
            pltpu.async_copy(h0_hbm.at[sidx[p]], rows[p], gsem[p])

        @pl.when(c == 1)
        def _():
            pltpu.async_copy(h1_hbm.at[sidx[p]], rows[p], gsem[p])

    def gwait(p):
        pltpu.make_async_copy(h0_hbm.at[sidx[p]], rows[p], gsem[p]).wait()

    def swait(p):
        pltpu.make_async_copy(rows[p], acc_sh.at[didx[p]], ssem[p]).wait()

    gath(0, 0)
    gath(1, 1)

    def group(g, _):
        for t in range(3):
            k3 = 3 * g + t
            pm1 = (t + 2) % 3
            gwait(t)
            pltpu.sync_copy(dst_hbm.at[pl.ds(tb + k3 * CH, CH)], didx[t])
            pltpu.async_copy(rows[t], acc_sh.at[didx[t]], ssem[t], add=True)
            if t == 0:
                @pl.when(g > 0)
                def _():
                    swait(pm1)
            else:
                swait(pm1)
            gath(pm1, k3 + 2)
        return 0
    lax.fori_loop(0, KE // 3, group, 0)
    gwait(0)
    gwait(1)
    swait(2)
    plsc.subcore_barrier()
    pltpu.sync_copy(acc_sh.at[pl.ds(s * RT, RT)],
                    out_hbm.at[c, pl.ds(s * RT, RT)])


def _edge_pass(h0, h1, src_pad, dst_pad, zeros_np):
    return _edge_kernel(h0, h1, src_pad, dst_pad, zeros_np)


def kernel(elements, oxidations, geometries, angles, edge_index, batch,
           emb_elem, emb_ox, emb_geo,
           gcn_w1, gcn_b1, gcn_w2, gcn_b2, gcn_w3, gcn_b3,
           dense_w1, dense_b1, dense_w2, dense_b2, dense_w3, dense_b3):
    elem3 = elements.astype(jnp.int32).reshape(G, R, 1)
    ox3 = oxidations.astype(jnp.int32).reshape(G, R, 1)
    geo3 = geometries.astype(jnp.int32).reshape(G, R, 1)
    batch3 = batch.astype(jnp.int32).reshape(G, R, 1)
    angles = angles.astype(jnp.float32)

    w1p = jnp.pad(gcn_w1, ((0, 34), (0, 34)))
    b2d = jnp.zeros((CIN, F), jnp.float32)
    b2d = b2d.at[:NUM_ELEMENTS, :200].set(emb_elem)
    b2d = b2d.at[NUM_ELEMENTS:116, 200:210].set(emb_ox)
    b2d = b2d.at[116:180, 210:220].set(emb_geo)
    b2d = b2d.at[180, 220].set(1.0).at[181, 221].set(1.0)
    w1eff = b2d @ w1p

    w2p = jnp.pad(gcn_w2, ((0, 34), (0, 34)))
    w3p = jnp.pad(gcn_w3, ((0, 34), (0, 34)))
    b1p = jnp.pad(gcn_b1, (0, 34)).reshape(1, F)
    b2p = jnp.pad(gcn_b2, (0, 34)).reshape(1, F)
    b3p = jnp.pad(gcn_b3, (0, 34)).reshape(1, F)
    dw1p = jnp.pad(dense_w1, ((0, 34), (0, 0)))
    db1 = dense_b1.reshape(1, 512)
    db2 = dense_b2.reshape(1, 128)
    dw3p = jnp.pad(dense_w3, ((0, 0), (0, 7)))
    db3p = jnp.pad(dense_b3, (0, 7)).reshape(1, 8)

    src_pad = jnp.full((EP + 2 * CH,), 0, jnp.int32)
    src_pad = src_pad.at[:E].set(edge_index[0].astype(jnp.int32))
    src_pad = src_pad.at[E:E + N].set(jnp.arange(N, dtype=jnp.int32))
    dst_pad = N + (jnp.arange(EP + 2 * CH, dtype=jnp.int32) % (NP - N))
    dst_pad = dst_pad.at[:E].set(edge_index[1].astype(jnp.int32))
    dst_pad = dst_pad.at[E:E + N].set(jnp.arange(N, dtype=jnp.int32))
    zeros_np = jnp.zeros((NP, FH), jnp.float32)

    deg2 = _deg_pass(src_pad, dst_pad)
    dinv8, h0, h1 = _mat1(elem3, ox3, geo3, angles, deg2, w1eff)
    acc2 = _edge_pass(h0, h1, src_pad, dst_pad, zeros_np)
    h0, h1 = _mat2(acc2, dinv8, b1p, w2p)
    acc2 = _edge_pass(h0, h1, src_pad, dst_pad, zeros_np)
    h0, h1 = _mat2(acc2, dinv8, b2p, w3p)
    acc2 = _edge_pass(h0, h1, src_pad, dst_pad, zeros_np)
    out8 = _head(acc2, dinv8, b3p, batch3, dw1p, db1, dense_w2, db2, dw3p, db3p)
    return out8[:, :1]


# speedup vs baseline: 2.8579x; 1.0675x over previous
import functools

import jax
import jax.numpy as jnp
from jax import lax
from jax.experimental import pallas as pl
from jax.experimental.pallas import tpu as pltpu
from jax.experimental.pallas import tpu_sc as plsc

N = 10000
E = 320000
NUM_GRAPHS = 256
NUM_ELEMENTS = 100
NUM_OXIDATIONS = 16
NUM_GEOMETRIES = 64
F = 256
FH = 128
NP = 10112
R = 1000
G = N // R
CIN = 184


def _elu(x):
    return jnp.where(x > 0, x, jnp.exp(jnp.minimum(x, 0.0)) - 1.0)


def _mat1_body(elem_ref, ox_ref, geo_ref, ang_ref, deg_ref, w_ref,
               dinv_ref, h0_ref, h1_ref):
    cls = lax.broadcasted_iota(jnp.int32, (1, CIN), 1)
    oh = ((elem_ref[0] == cls).astype(jnp.float32)
          + (ox_ref[0] + NUM_ELEMENTS == cls).astype(jnp.float32)
          + (geo_ref[0] + NUM_ELEMENTS + NUM_OXIDATIONS == cls).astype(jnp.float32))
    a0 = ang_ref[:, 0:1] * (cls == 180).astype(jnp.float32)
    a1 = ang_ref[:, 1:2] * (cls == 181).astype(jnp.float32)
    x_in = oh + a0 + a1
    deg = deg_ref[0][:, 0:1] + deg_ref[1][:, 0:1]
    dinv = lax.rsqrt(deg)
    h = jnp.dot(x_in, w_ref[...], preferred_element_type=jnp.float32) * dinv
    dinv_ref[...] = jnp.broadcast_to(dinv, (R, 8))
    h0_ref[...] = h[:, :FH]
    h1_ref[...] = h[:, FH:]


def _mat1(elem3, ox3, geo3, angles, deg2, w1eff):
    return pl.pallas_call(
        _mat1_body,
        grid=(G,),
        in_specs=[
            pl.BlockSpec((1, R, 1), lambda i: (i, 0, 0)),
            pl.BlockSpec((1, R, 1), lambda i: (i, 0, 0)),
            pl.BlockSpec((1, R, 1), lambda i: (i, 0, 0)),
            pl.BlockSpec((R, 2), lambda i: (i, 0)),
            pl.BlockSpec((2, R, 16), lambda i: (0, i, 0)),
            pl.BlockSpec((CIN, F), lambda i: (0, 0)),
        ],
        out_specs=[
            pl.BlockSpec((R, 8), lambda i: (i, 0)),
            pl.BlockSpec((R, FH), lambda i: (i, 0)),
            pl.BlockSpec((R, FH), lambda i: (i, 0)),
        ],
        out_shape=[
            jax.ShapeDtypeStruct((N, 8), jnp.float32),
            jax.ShapeDtypeStruct((N, FH), jnp.float32),
            jax.ShapeDtypeStruct((N, FH), jnp.float32),
        ],
    )(elem3, ox3, geo3, angles, deg2, w1eff)


def _mat2_body(acc_ref, dinv_ref, b_ref, w_ref, h0_ref, h1_ref):
    xcat = jnp.concatenate([acc_ref[0], acc_ref[1]], axis=1)
    dinv = dinv_ref[:, 0:1]
    x = _elu(xcat * dinv + b_ref[...])
    h = jnp.dot(x, w_ref[...], preferred_element_type=jnp.float32) * dinv
    h0_ref[...] = h[:, :FH]
    h1_ref[...] = h[:, FH:]


def _mat2(acc2, dinv8, bprev, w):
    return pl.pallas_call(
        _mat2_body,
        grid=(G,),
        in_specs=[
            pl.BlockSpec((2, R, FH), lambda i: (0, i, 0)),
            pl.BlockSpec((R, 8), lambda i: (i, 0)),
            pl.BlockSpec((1, F), lambda i: (0, 0)),
            pl.BlockSpec((F, F), lambda i: (0, 0)),
        ],
        out_specs=[
            pl.BlockSpec((R, FH), lambda i: (i, 0)),
            pl.BlockSpec((R, FH), lambda i: (i, 0)),
        ],
        out_shape=[
            jax.ShapeDtypeStruct((N, FH), jnp.float32),
            jax.ShapeDtypeStruct((N, FH), jnp.float32),
        ],
    )(acc2, dinv8, bprev, w)


def _head_body(acc_ref, dinv_ref, b3_ref, batch_ref, dw1_ref, db1_ref,
               dw2_ref, db2_ref, dw3_ref, db3_ref, out_ref, sums_ref, cnt_ref):
    i = pl.program_id(0)
    xcat = jnp.concatenate([acc_ref[0], acc_ref[1]], axis=1)
    x3 = _elu(xcat * dinv_ref[:, 0:1] + b3_ref[...])
    gcls = lax.broadcasted_iota(jnp.int32, (1, NUM_GRAPHS), 1)
    oh = (batch_ref[0] == gcls).astype(jnp.float32)
    dn = (((0,), (0,)), ((), ()))
    s = lax.dot_general(oh, x3, dn, preferred_element_type=jnp.float32)
    c = lax.dot_general(oh, jnp.ones((R, 8), jnp.float32), dn,
                        preferred_element_type=jnp.float32)

    @pl.when(i == 0)
    def _():
        sums_ref[...] = s
        cnt_ref[...] = c

    @pl.when(i > 0)
    def _():
        sums_ref[...] += s
        cnt_ref[...] += c

    @pl.when(i == G - 1)
    def _():
        pooled = sums_ref[...] / jnp.maximum(cnt_ref[:, 0:1], 1.0)
        h1 = _elu(jnp.dot(pooled, dw1_ref[...],
                          preferred_element_type=jnp.float32) + db1_ref[...])
        h2 = _elu(jnp.dot(h1, dw2_ref[...],
                          preferred_element_type=jnp.float32) + db2_ref[...])
        out_ref[...] = jnp.dot(h2, dw3_ref[...],
                               preferred_element_type=jnp.float32) + db3_ref[...]


def _head(acc2, dinv8, b3, batch3, dw1, db1, dw2, db2, dw3p, db3p):
    return pl.pallas_call(
        _head_body,
        grid=(G,),
        in_specs=[
            pl.BlockSpec((2, R, FH), lambda i: (0, i, 0)),
            pl.BlockSpec((R, 8), lambda i: (i, 0)),
            pl.BlockSpec((1, F), lambda i: (0, 0)),
            pl.BlockSpec((1, R, 1), lambda i: (i, 0, 0)),
            pl.BlockSpec((F, 512), lambda i: (0, 0)),
            pl.BlockSpec((1, 512), lambda i: (0, 0)),
            pl.BlockSpec((512, 128), lambda i: (0, 0)),
            pl.BlockSpec((1, 128), lambda i: (0, 0)),
            pl.BlockSpec((128, 8), lambda i: (0, 0)),
            pl.BlockSpec((1, 8), lambda i: (0, 0)),
        ],
        out_specs=pl.BlockSpec((NUM_GRAPHS, 8), lambda i: (0, 0)),
        out_shape=jax.ShapeDtypeStruct((NUM_GRAPHS, 8), jnp.float32),
        scratch_shapes=[
            pltpu.VMEM((NUM_GRAPHS, F), jnp.float32),
            pltpu.VMEM((NUM_GRAPHS, 8), jnp.float32),
        ],
    )(acc2, dinv8, b3, batch3, dw1, db1, dw2, db2, dw3p, db3p)


EP = 331776
CH = 128
TE = EP // 16
KE = TE // CH
TD = EP // 32
KD = TD // CH
RT = NP // 16
_SC_MESH = dict(core_axis_name="c", subcore_axis_name="s")


@functools.partial(
    pl.kernel,
    mesh=plsc.VectorSubcoreMesh(**_SC_MESH),
    out_type=jax.ShapeDtypeStruct((2, NP, 16), jnp.float32),
    scratch_types=[
        pltpu.VMEM_SHARED((NP, 16), jnp.float32),
        pltpu.VMEM((RT, 16), jnp.float32),
        pltpu.VMEM((CH, 16), jnp.float32),
        pltpu.VMEM((CH,), jnp.int32),
    ],
)
def _deg_kernel(dst_hbm, out_hbm, acc_sh, zbuf, ones, didx):
    c = lax.axis_index("c")
    s = lax.axis_index("s")

    def zrow(j, _):
        zbuf[j, :] = jnp.zeros((16,), jnp.float32)
        return 0
    lax.fori_loop(0, RT, zrow, 0)
    pltpu.sync_copy(zbuf, acc_sh.at[pl.ds(s * RT, RT)])

    def orow(j, _):
        ones[j, :] = jnp.full((16,), 1.0, jnp.float32)
        return 0
    lax.fori_loop(0, CH, orow, 0)
    plsc.subcore_barrier()

    w = c * 16 + s

    def body(k, _):
        base = w * TD + k * CH
        pltpu.sync_copy(dst_hbm.at[pl.ds(base, CH)], didx)
        pltpu.sync_copy(ones, acc_sh.at[didx], add=True)
        return 0
    lax.fori_loop(0, KD, body, 0)
    plsc.subcore_barrier()
    pltpu.sync_copy(acc_sh.at[pl.ds(s * RT, RT)],
                    out_hbm.at[c, pl.ds(s * RT, RT)])


def _deg_pass(src_pad, dst_pad):
    return _deg_kernel(dst_pad)


@functools.partial(
    pl.kernel,
    mesh=plsc.VectorSubcoreMesh(**_SC_MESH),
    out_type=jax.ShapeDtypeStruct((2, NP, FH), jnp.float32),
    scratch_types=[
        pltpu.VMEM_SHARED((NP, FH), jnp.float32),
        pltpu.VMEM((CH, FH), jnp.float32),
        pltpu.VMEM((CH, FH), jnp.float32),
        pltpu.VMEM((CH, FH), jnp.float32),
        pltpu.VMEM((CH,), jnp.int32),
        pltpu.VMEM((CH,), jnp.int32),
        pltpu.VMEM((CH,), jnp.int32),
        pltpu.VMEM((CH,), jnp.int32),
        pltpu.VMEM((CH,), jnp.int32),
        pltpu.VMEM((CH,), jnp.int32),
        pltpu.SemaphoreType.DMA,
        pltpu.SemaphoreType.DMA,
        pltpu.SemaphoreType.DMA,
        pltpu.SemaphoreType.DMA,
        pltpu.SemaphoreType.DMA,
        pltpu.SemaphoreType.DMA,
        pltpu.SemaphoreType.DMA,
        pltpu.SemaphoreType.DMA,
        pltpu.SemaphoreType.DMA,
        pltpu.SemaphoreType.DMA,
        pltpu.SemaphoreType.DMA,
        pltpu.SemaphoreType.DMA,
    ],
)
def _edge_kernel(h0_hbm, h1_hbm, src_hbm, dst_hbm, zero_hbm, out_hbm,
                 acc_sh, rows0, rows1, rows2,
                 sidx0, sidx1, sidx2, didx0, didx1, didx2,
                 gsem0, gsem1, gsem2, ssem0, ssem1, ssem2,
                 isem0, isem1, isem2, dsem0, dsem1, dsem2):
    c = lax.axis_index("c")
    s = lax.axis_index("s")
    rows = (rows0, rows1, rows2)
    sidx = (sidx0, sidx1, sidx2)
    didx = (didx0, didx1, didx2)
    gsem = (gsem0, gsem1, gsem2)
    ssem = (ssem0, ssem1, ssem2)
    isem = (isem0, isem1, isem2)
    dsem = (dsem0, dsem1, dsem2)

    pltpu.sync_copy(zero_hbm.at[pl.ds(s * RT, RT)],
                    acc_sh.at[pl.ds(s * RT, RT)])
    plsc.subcore_barrier()

    tb = s * TE

    def sfetch(p, k):
        pltpu.async_copy(src_hbm.at[pl.ds(tb + k * CH, CH)], sidx[p], isem[p])

    def dfetch(p, k):
        pltpu.async_copy(dst_hbm.at[pl.ds(tb + k * CH, CH)], didx[p], dsem[p])

    def iwait(p):
        pltpu.make_async_copy(src_hbm.at[pl.ds(tb, CH)], sidx[p],
                              isem[p]).wait()

    def dwait(p):
        pltpu.make_async_copy(dst_hbm.at[pl.ds(tb, CH)], didx[p],
                              dsem[p]).wait()

    def gath(p):
        @pl.when(c == 0)
        def _():
            pltpu.async_copy(h0_hbm.at[sidx[p]], rows[p], gsem[p])

        @pl.when(c == 1)
        def _():
            pltpu.async_copy(h1_hbm.at[sidx[p]], rows[p], gsem[p])

    def gwait(p):
        pltpu.make_async_copy(h0_hbm.at[sidx[p]], rows[p], gsem[p]).wait()

    def swait(p):
        pltpu.make_async_copy(rows[p], acc_sh.at[didx[p]], ssem[p]).wait()

    sfetch(0, 0)
    sfetch(1, 1)
    sfetch(2, 2)
    dfetch(0, 0)
    dfetch(1, 1)
    iwait(0)
    gath(0)
    iwait(1)
    gath(1)

    def group(g, _):
        for t in range(3):
            k3 = 3 * g + t
            pn = (t + 2) % 3
            gwait(t)
            sfetch(t, k3 + 3)
            dwait(t)
            pltpu.async_copy(rows[t], acc_sh.at[didx[t]], ssem[t], add=True)
            if t == 0:
                @pl.when(g > 0)
                def _():
                    swait(pn)
            else:
                swait(pn)
            dfetch(pn, k3 + 2)
            iwait(pn)
            gath(pn)
        return 0
    lax.fori_loop(0, KE // 3, group, 0)
    gwait(0)
    gwait(1)
    swait(2)
    iwait(2)
    dwait(0)
    dwait(1)
    plsc.subcore_barrier()
    pltpu.sync_copy(acc_sh.at[pl.ds(s * RT, RT)],
                    out_hbm.at[c, pl.ds(s * RT, RT)])


def _edge_pass(h0, h1, src_pad, dst_pad, zeros_np):
    return _edge_kernel(h0, h1, src_pad, dst_pad, zeros_np)


def kernel(elements, oxidations, geometries, angles, edge_index, batch,
           emb_elem, emb_ox, emb_geo,
           gcn_w1, gcn_b1, gcn_w2, gcn_b2, gcn_w3, gcn_b3,
           dense_w1, dense_b1, dense_w2, dense_b2, dense_w3, dense_b3):
    elem3 = elements.astype(jnp.int32).reshape(G, R, 1)
    ox3 = oxidations.astype(jnp.int32).reshape(G, R, 1)
    geo3 = geometries.astype(jnp.int32).reshape(G, R, 1)
    batch3 = batch.astype(jnp.int32).reshape(G, R, 1)
    angles = angles.astype(jnp.float32)

    w1p = jnp.pad(gcn_w1, ((0, 34), (0, 34)))
    b2d = jnp.zeros((CIN, F), jnp.float32)
    b2d = b2d.at[:NUM_ELEMENTS, :200].set(emb_elem)
    b2d = b2d.at[NUM_ELEMENTS:116, 200:210].set(emb_ox)
    b2d = b2d.at[116:180, 210:220].set(emb_geo)
    b2d = b2d.at[180, 220].set(1.0).at[181, 221].set(1.0)
    w1eff = b2d @ w1p

    w2p = jnp.pad(gcn_w2, ((0, 34), (0, 34)))
    w3p = jnp.pad(gcn_w3, ((0, 34), (0, 34)))
    b1p = jnp.pad(gcn_b1, (0, 34)).reshape(1, F)
    b2p = jnp.pad(gcn_b2, (0, 34)).reshape(1, F)
    b3p = jnp.pad(gcn_b3, (0, 34)).reshape(1, F)
    dw1p = jnp.pad(dense_w1, ((0, 34), (0, 0)))
    db1 = dense_b1.reshape(1, 512)
    db2 = dense_b2.reshape(1, 128)
    dw3p = jnp.pad(dense_w3, ((0, 0), (0, 7)))
    db3p = jnp.pad(dense_b3, (0, 7)).reshape(1, 8)

    src_pad = jnp.full((EP + 3 * CH,), 0, jnp.int32)
    src_pad = src_pad.at[:E].set(edge_index[0].astype(jnp.int32))
    src_pad = src_pad.at[E:E + N].set(jnp.arange(N, dtype=jnp.int32))
    dst_pad = N + (jnp.arange(EP + 2 * CH, dtype=jnp.int32) % (NP - N))
    dst_pad = dst_pad.at[:E].set(edge_index[1].astype(jnp.int32))
    dst_pad = dst_pad.at[E:E + N].set(jnp.arange(N, dtype=jnp.int32))
    zeros_np = jnp.zeros((NP, FH), jnp.float32)

    deg2 = _deg_pass(src_pad, dst_pad)
    dinv8, h0, h1 = _mat1(elem3, ox3, geo3, angles, deg2, w1eff)
    acc2 = _edge_pass(h0, h1, src_pad, dst_pad, zeros_np)
    h0, h1 = _mat2(acc2, dinv8, b1p, w2p)
    acc2 = _edge_pass(h0, h1, src_pad, dst_pad, zeros_np)
    h0, h1 = _mat2(acc2, dinv8, b2p, w3p)
    acc2 = _edge_pass(h0, h1, src_pad, dst_pad, zeros_np)
    out8 = _head(acc2, dinv8, b3p, batch3, dw1p, db1, dense_w2, db2, dw3p, db3p)
    return out8[:, :1]


# speedup vs baseline: 2.9700x; 1.0392x over previous
import functools

import jax
import jax.numpy as jnp
from jax import lax
from jax.experimental import pallas as pl
from jax.experimental.pallas import tpu as pltpu
from jax.experimental.pallas import tpu_sc as plsc

N = 10000
E = 320000
NUM_GRAPHS = 256
NUM_ELEMENTS = 100
NUM_OXIDATIONS = 16
NUM_GEOMETRIES = 64
F = 256
FH = 128
NP = 10112
R = 1000
G = N // R
CIN = 184


def _elu(x):
    return jnp.where(x > 0, x, jnp.exp(jnp.minimum(x, 0.0)) - 1.0)


def _mat1_body(elem_ref, ox_ref, geo_ref, ang_ref, deg_ref, w_ref,
               dinv_ref, h0_ref, h1_ref):
    cls = lax.broadcasted_iota(jnp.int32, (1, CIN), 1)
    oh = ((elem_ref[0] == cls).astype(jnp.float32)
          + (ox_ref[0] + NUM_ELEMENTS == cls).astype(jnp.float32)
          + (geo_ref[0] + NUM_ELEMENTS + NUM_OXIDATIONS == cls).astype(jnp.float32))
    a0 = ang_ref[:, 0:1] * (cls == 180).astype(jnp.float32)
    a1 = ang_ref[:, 1:2] * (cls == 181).astype(jnp.float32)
    x_in = oh + a0 + a1
    deg = deg_ref[0][:, 0:1] + deg_ref[1][:, 0:1]
    dinv = lax.rsqrt(deg)
    h = jnp.dot(x_in, w_ref[...], preferred_element_type=jnp.float32) * dinv
    dinv_ref[...] = jnp.broadcast_to(dinv, (R, 8))
    h0_ref[...] = h[:, :FH]
    h1_ref[...] = h[:, FH:]


def _mat1(elem3, ox3, geo3, angles, deg2, w1eff):
    return pl.pallas_call(
        _mat1_body,
        grid=(G,),
        in_specs=[
            pl.BlockSpec((1, R, 1), lambda i: (i, 0, 0)),
            pl.BlockSpec((1, R, 1), lambda i: (i, 0, 0)),
            pl.BlockSpec((1, R, 1), lambda i: (i, 0, 0)),
            pl.BlockSpec((R, 2), lambda i: (i, 0)),
            pl.BlockSpec((2, R, 16), lambda i: (0, i, 0)),
            pl.BlockSpec((CIN, F), lambda i: (0, 0)),
        ],
        out_specs=[
            pl.BlockSpec((R, 8), lambda i: (i, 0)),
            pl.BlockSpec((R, FH), lambda i: (i, 0)),
            pl.BlockSpec((R, FH), lambda i: (i, 0)),
        ],
        out_shape=[
            jax.ShapeDtypeStruct((N, 8), jnp.float32),
            jax.ShapeDtypeStruct((N, FH), jnp.float32),
            jax.ShapeDtypeStruct((N, FH), jnp.float32),
        ],
    )(elem3, ox3, geo3, angles, deg2, w1eff)


def _mat2_body(acc_ref, dinv_ref, b_ref, w_ref, h0_ref, h1_ref):
    xcat = jnp.concatenate([acc_ref[0], acc_ref[1]], axis=1)
    dinv = dinv_ref[:, 0:1]
    x = _elu(xcat * dinv + b_ref[...])
    h = jnp.dot(x, w_ref[...], preferred_element_type=jnp.float32) * dinv
    h0_ref[...] = h[:, :FH]
    h1_ref[...] = h[:, FH:]


def _mat2(acc2, dinv8, bprev, w):
    return pl.pallas_call(
        _mat2_body,
        grid=(G,),
        in_specs=[
            pl.BlockSpec((2, R, FH), lambda i: (0, i, 0)),
            pl.BlockSpec((R, 8), lambda i: (i, 0)),
            pl.BlockSpec((1, F), lambda i: (0, 0)),
            pl.BlockSpec((F, F), lambda i: (0, 0)),
        ],
        out_specs=[
            pl.BlockSpec((R, FH), lambda i: (i, 0)),
            pl.BlockSpec((R, FH), lambda i: (i, 0)),
        ],
        out_shape=[
            jax.ShapeDtypeStruct((N, FH), jnp.float32),
            jax.ShapeDtypeStruct((N, FH), jnp.float32),
        ],
    )(acc2, dinv8, bprev, w)


def _head_body(acc_ref, dinv_ref, b3_ref, batch_ref, dw1_ref, db1_ref,
               dw2_ref, db2_ref, dw3_ref, db3_ref, out_ref, sums_ref, cnt_ref):
    i = pl.program_id(0)
    xcat = jnp.concatenate([acc_ref[0], acc_ref[1]], axis=1)
    x3 = _elu(xcat * dinv_ref[:, 0:1] + b3_ref[...])
    gcls = lax.broadcasted_iota(jnp.int32, (1, NUM_GRAPHS), 1)
    oh = (batch_ref[0] == gcls).astype(jnp.float32)
    dn = (((0,), (0,)), ((), ()))
    s = lax.dot_general(oh, x3, dn, preferred_element_type=jnp.float32)
    c = lax.dot_general(oh, jnp.ones((R, 8), jnp.float32), dn,
                        preferred_element_type=jnp.float32)

    @pl.when(i == 0)
    def _():
        sums_ref[...] = s
        cnt_ref[...] = c

    @pl.when(i > 0)
    def _():
        sums_ref[...] += s
        cnt_ref[...] += c

    @pl.when(i == G - 1)
    def _():
        pooled = sums_ref[...] / jnp.maximum(cnt_ref[:, 0:1], 1.0)
        h1 = _elu(jnp.dot(pooled, dw1_ref[...],
                          preferred_element_type=jnp.float32) + db1_ref[...])
        h2 = _elu(jnp.dot(h1, dw2_ref[...],
                          preferred_element_type=jnp.float32) + db2_ref[...])
        out_ref[...] = jnp.dot(h2, dw3_ref[...],
                               preferred_element_type=jnp.float32) + db3_ref[...]


def _head(acc2, dinv8, b3, batch3, dw1, db1, dw2, db2, dw3p, db3p):
    return pl.pallas_call(
        _head_body,
        grid=(G,),
        in_specs=[
            pl.BlockSpec((2, R, FH), lambda i: (0, i, 0)),
            pl.BlockSpec((R, 8), lambda i: (i, 0)),
            pl.BlockSpec((1, F), lambda i: (0, 0)),
            pl.BlockSpec((1, R, 1), lambda i: (i, 0, 0)),
            pl.BlockSpec((F, 512), lambda i: (0, 0)),
            pl.BlockSpec((1, 512), lambda i: (0, 0)),
            pl.BlockSpec((512, 128), lambda i: (0, 0)),
            pl.BlockSpec((1, 128), lambda i: (0, 0)),
            pl.BlockSpec((128, 8), lambda i: (0, 0)),
            pl.BlockSpec((1, 8), lambda i: (0, 0)),
        ],
        out_specs=pl.BlockSpec((NUM_GRAPHS, 8), lambda i: (0, 0)),
        out_shape=jax.ShapeDtypeStruct((NUM_GRAPHS, 8), jnp.float32),
        scratch_shapes=[
            pltpu.VMEM((NUM_GRAPHS, F), jnp.float32),
            pltpu.VMEM((NUM_GRAPHS, 8), jnp.float32),
        ],
    )(acc2, dinv8, b3, batch3, dw1, db1, dw2, db2, dw3p, db3p)


EP = 331776
CH = 128
TE = EP // 16
KE = TE // CH
KD = 84
EPD = 32 * KD * CH
RT = NP // 16
_SC_MESH = dict(core_axis_name="c", subcore_axis_name="s")


@functools.partial(
    pl.kernel,
    mesh=plsc.VectorSubcoreMesh(**_SC_MESH),
    out_type=jax.ShapeDtypeStruct((2, NP, 16), jnp.float32),
    scratch_types=[
        pltpu.VMEM_SHARED((NP, 16), jnp.float32),
        pltpu.VMEM((RT, 16), jnp.float32),
        pltpu.VMEM((CH, 16), jnp.float32),
        pltpu.VMEM((CH,), jnp.int32),
        pltpu.VMEM((CH,), jnp.int32),
        pltpu.VMEM((CH,), jnp.int32),
        pltpu.SemaphoreType.DMA,
        pltpu.SemaphoreType.DMA,
        pltpu.SemaphoreType.DMA,
        pltpu.SemaphoreType.DMA,
        pltpu.SemaphoreType.DMA,
        pltpu.SemaphoreType.DMA,
    ],
)
def _deg_kernel(dst_hbm, out_hbm, acc_sh, zbuf, ones,
                didx0, didx1, didx2, dsem0, dsem1, dsem2,
                ssem0, ssem1, ssem2):
    c = lax.axis_index("c")
    s = lax.axis_index("s")
    didx = (didx0, didx1, didx2)
    dsem = (dsem0, dsem1, dsem2)
    ssem = (ssem0, ssem1, ssem2)

    def zrow(j, _):
        zbuf[j, :] = jnp.zeros((16,), jnp.float32)
        return 0
    lax.fori_loop(0, RT, zrow, 0)
    pltpu.sync_copy(zbuf, acc_sh.at[pl.ds(s * RT, RT)])

    def orow(j, _):
        ones[j, :] = jnp.full((16,), 1.0, jnp.float32)
        return 0
    lax.fori_loop(0, CH, orow, 0)
    plsc.subcore_barrier()

    wb = (c * 16 + s) * KD

    def dfetch(p, k):
        pltpu.async_copy(dst_hbm.at[pl.ds((wb + k) * CH, CH)], didx[p],
                         dsem[p])

    def dwait(p):
        pltpu.make_async_copy(dst_hbm.at[pl.ds(wb * CH, CH)], didx[p],
                              dsem[p]).wait()

    def swait(p):
        pltpu.make_async_copy(ones, acc_sh.at[didx[p]], ssem[p]).wait()

    dfetch(0, 0)
    dfetch(1, 1)

    def group(g, _):
        for t in range(3):
            k3 = 3 * g + t
            pn = (t + 2) % 3
            dwait(t)
            pltpu.async_copy(ones, acc_sh.at[didx[t]], ssem[t], add=True)
            if t == 0:
                @pl.when(g > 0)
                def _():
                    swait(pn)
            else:
                swait(pn)
            dfetch(pn, k3 + 2)
        return 0
    lax.fori_loop(0, KD // 3, group, 0)
    swait(2)
    dwait(0)
    dwait(1)
    plsc.subcore_barrier()
    pltpu.sync_copy(acc_sh.at[pl.ds(s * RT, RT)],
                    out_hbm.at[c, pl.ds(s * RT, RT)])


def _deg_pass(src_pad, dst_pad):
    return _deg_kernel(dst_pad)


@functools.partial(
    pl.kernel,
    mesh=plsc.VectorSubcoreMesh(**_SC_MESH),
    out_type=jax.ShapeDtypeStruct((2, NP, FH), jnp.float32),
    scratch_types=[
        pltpu.VMEM_SHARED((NP, FH), jnp.float32),
        pltpu.VMEM((CH, FH), jnp.float32),
        pltpu.VMEM((CH, FH), jnp.float32),
        pltpu.VMEM((CH, FH), jnp.float32),
        pltpu.VMEM((CH,), jnp.int32),
        pltpu.VMEM((CH,), jnp.int32),
        pltpu.VMEM((CH,), jnp.int32),
        pltpu.VMEM((CH,), jnp.int32),
        pltpu.VMEM((CH,), jnp.int32),
        pltpu.VMEM((CH,), jnp.int32),
        pltpu.SemaphoreType.DMA,
        pltpu.SemaphoreType.DMA,
        pltpu.SemaphoreType.DMA,
        pltpu.SemaphoreType.DMA,
        pltpu.SemaphoreType.DMA,
        pltpu.SemaphoreType.DMA,
        pltpu.SemaphoreType.DMA,
        pltpu.SemaphoreType.DMA,
        pltpu.SemaphoreType.DMA,
        pltpu.SemaphoreType.DMA,
        pltpu.SemaphoreType.DMA,
        pltpu.SemaphoreType.DMA,
    ],
)
def _edge_kernel(h0_hbm, h1_hbm, src_hbm, dst_hbm, zero_hbm, out_hbm,
                 acc_sh, rows0, rows1, rows2,
                 sidx0, sidx1, sidx2, didx0, didx1, didx2,
                 gsem0, gsem1, gsem2, ssem0, ssem1, ssem2,
                 isem0, isem1, isem2, dsem0, dsem1, dsem2):
    c = lax.axis_index("c")
    s = lax.axis_index("s")
    rows = (rows0, rows1, rows2)
    sidx = (sidx0, sidx1, sidx2)
    didx = (didx0, didx1, didx2)
    gsem = (gsem0, gsem1, gsem2)
    ssem = (ssem0, ssem1, ssem2)
    isem = (isem0, isem1, isem2)
    dsem = (dsem0, dsem1, dsem2)

    pltpu.sync_copy(zero_hbm.at[pl.ds(s * RT, RT)],
                    acc_sh.at[pl.ds(s * RT, RT)])
    plsc.subcore_barrier()

    tb = s * TE

    def sfetch(p, k):
        pltpu.async_copy(src_hbm.at[pl.ds(tb + k * CH, CH)], sidx[p], isem[p])

    def dfetch(p, k):
        pltpu.async_copy(dst_hbm.at[pl.ds(tb + k * CH, CH)], didx[p], dsem[p])

    def iwait(p):
        pltpu.make_async_copy(src_hbm.at[pl.ds(tb, CH)], sidx[p],
                              isem[p]).wait()

    def dwait(p):
        pltpu.make_async_copy(dst_hbm.at[pl.ds(tb, CH)], didx[p],
                              dsem[p]).wait()

    def gath(p):
        @pl.when(c == 0)
        def _():
            pltpu.async_copy(h0_hbm.at[sidx[p]], rows[p], gsem[p])

        @pl.when(c == 1)
        def _():
            pltpu.async_copy(h1_hbm.at[sidx[p]], rows[p], gsem[p])

    def gwait(p):
        pltpu.make_async_copy(h0_hbm.at[sidx[p]], rows[p], gsem[p]).wait()

    def swait(p):
        pltpu.make_async_copy(rows[p], acc_sh.at[didx[p]], ssem[p]).wait()

    sfetch(0, 0)
    sfetch(1, 1)
    sfetch(2, 2)
    dfetch(0, 0)
    dfetch(1, 1)
    iwait(0)
    gath(0)
    iwait(1)
    gath(1)

    def group(g, _):
        for t in range(3):
            k3 = 3 * g + t
            pn = (t + 2) % 3
            gwait(t)
            sfetch(t, k3 + 3)
            dwait(t)
            pltpu.async_copy(rows[t], acc_sh.at[didx[t]], ssem[t], add=True)
            if t == 0:
                @pl.when(g > 0)
                def _():
                    swait(pn)
            else:
                swait(pn)
            dfetch(pn, k3 + 2)
            iwait(pn)
            gath(pn)
        return 0
    lax.fori_loop(0, KE // 3, group, 0)
    gwait(0)
    gwait(1)
    swait(2)
    iwait(2)
    dwait(0)
    dwait(1)
    plsc.subcore_barrier()
    pltpu.sync_copy(acc_sh.at[pl.ds(s * RT, RT)],
                    out_hbm.at[c, pl.ds(s * RT, RT)])


def _edge_pass(h0, h1, src_pad, dst_pad, zeros_np):
    return _edge_kernel(h0, h1, src_pad, dst_pad, zeros_np)


def kernel(elements, oxidations, geometries, angles, edge_index, batch,
           emb_elem, emb_ox, emb_geo,
           gcn_w1, gcn_b1, gcn_w2, gcn_b2, gcn_w3, gcn_b3,
           dense_w1, dense_b1, dense_w2, dense_b2, dense_w3, dense_b3):
    elem3 = elements.astype(jnp.int32).reshape(G, R, 1)
    ox3 = oxidations.astype(jnp.int32).reshape(G, R, 1)
    geo3 = geometries.astype(jnp.int32).reshape(G, R, 1)
    batch3 = batch.astype(jnp.int32).reshape(G, R, 1)
    angles = angles.astype(jnp.float32)

    w1p = jnp.pad(gcn_w1, ((0, 34), (0, 34)))
    b2d = jnp.zeros((CIN, F), jnp.float32)
    b2d = b2d.at[:NUM_ELEMENTS, :200].set(emb_elem)
    b2d = b2d.at[NUM_ELEMENTS:116, 200:210].set(emb_ox)
    b2d = b2d.at[116:180, 210:220].set(emb_geo)
    b2d = b2d.at[180, 220].set(1.0).at[181, 221].set(1.0)
    w1eff = b2d @ w1p

    w2p = jnp.pad(gcn_w2, ((0, 34), (0, 34)))
    w3p = jnp.pad(gcn_w3, ((0, 34), (0, 34)))
    b1p = jnp.pad(gcn_b1, (0, 34)).reshape(1, F)
    b2p = jnp.pad(gcn_b2, (0, 34)).reshape(1, F)
    b3p = jnp.pad(gcn_b3, (0, 34)).reshape(1, F)
    dw1p = jnp.pad(dense_w1, ((0, 34), (0, 0)))
    db1 = dense_b1.reshape(1, 512)
    db2 = dense_b2.reshape(1, 128)
    dw3p = jnp.pad(dense_w3, ((0, 0), (0, 7)))
    db3p = jnp.pad(dense_b3, (0, 7)).reshape(1, 8)

    src_pad = jnp.full((EP + 3 * CH,), 0, jnp.int32)
    src_pad = src_pad.at[:E].set(edge_index[0].astype(jnp.int32))
    src_pad = src_pad.at[E:E + N].set(jnp.arange(N, dtype=jnp.int32))
    dst_pad = N + (jnp.arange(EPD + 2 * CH, dtype=jnp.int32) % (NP - N))
    dst_pad = dst_pad.at[:E].set(edge_index[1].astype(jnp.int32))
    dst_pad = dst_pad.at[E:E + N].set(jnp.arange(N, dtype=jnp.int32))
    zeros_np = jnp.zeros((NP, FH), jnp.float32)

    deg2 = _deg_pass(src_pad, dst_pad)
    dinv8, h0, h1 = _mat1(elem3, ox3, geo3, angles, deg2, w1eff)
    acc2 = _edge_pass(h0, h1, src_pad, dst_pad, zeros_np)
    h0, h1 = _mat2(acc2, dinv8, b1p, w2p)
    acc2 = _edge_pass(h0, h1, src_pad, dst_pad, zeros_np)
    h0, h1 = _mat2(acc2, dinv8, b2p, w3p)
    acc2 = _edge_pass(h0, h1, src_pad, dst_pad, zeros_np)
    out8 = _head(acc2, dinv8, b3p, batch3, dw1p, db1, dense_w2, db2, dw3p, db3p)
    return out8[:, :1]
